# Initial kernel scaffold; baseline (speedup 1.0000x reference)
import jax
import jax.numpy as jnp
from jax.experimental import pallas as pl


def kernel(pt_coord, feats, W1, g1, b1, W2, g2, b2, W3, g3, b3, W4, g4, b4, W5, g5, b5, Wf1, gf1, bf1, Wf2, gf2, bf2, Wf3, gf3, bf3, Wsem, bsem):
    raise NotImplementedError("write your pallas kernel here")


# speedup vs baseline: 2.7151x; 2.7151x over previous
import functools

import jax
import jax.numpy as jnp
import numpy as np
from jax import lax
from jax.experimental import pallas as pl
from jax.experimental.pallas import tpu as pltpu
from jax.experimental.pallas import tpu_sc as plsc

K = 20
EPS = 1e-5
NEG = -3.0e38

NC = 2
NS = 16
NW = NC * NS
IDX_PER_ROW = 80

_HI = lax.Precision.HIGHEST


def _topk_body(R, n, xblk_ref, xfull_ref, idx_ref):
    xb = xblk_ref[...]
    xf = xfull_ref[...]
    xb16 = xb.astype(jnp.bfloat16)
    xf16 = xf.astype(jnp.bfloat16)
    t = lax.dot_general(xb16, xf16, (((1,), (1,)), ((), ())),
                        preferred_element_type=jnp.float32)
    inner = -2.0 * t
    xxr = jnp.sum(xb * xb, axis=1, keepdims=True)
    xsq = xf * xf
    onesc = jnp.ones((1, xf.shape[1]), jnp.float32)
    xxc = lax.dot_general(onesc, xsq, (((1,), (1,)), ((), ())),
                          precision=_HI,
                          preferred_element_type=jnp.float32)
    score = -xxr - inner - xxc
    kcols = lax.broadcasted_iota(jnp.int32, (R, K), 1)

    def it(k, carry):
        s, acc = carry
        cols = lax.broadcasted_iota(jnp.int32, (R, n), 1)
        m = jnp.max(s, axis=1, keepdims=True)
        cand = jnp.where(s >= m, cols, n)
        fi = jnp.min(cand, axis=1, keepdims=True)
        acc = jnp.where(kcols == k, fi, acc)
        s = jnp.where(cols == fi, NEG, s)
        return s, acc

    _, acc = lax.fori_loop(0, K, it, (score, jnp.zeros((R, K), jnp.int32)))
    idx_ref[...] = acc


def _topk(X):
    n, C = X.shape
    R = 200 if n % 200 == 0 else 8
    grid = (n // R,)
    return pl.pallas_call(
        functools.partial(_topk_body, R, n),
        grid=grid,
        in_specs=[pl.BlockSpec((R, C), lambda i: (i, 0)),
                  pl.BlockSpec((n, C), lambda i: (0, 0))],
        out_specs=pl.BlockSpec((R, K), lambda i: (i, 0)),
        out_shape=jax.ShapeDtypeStruct((n, K), jnp.int32),
    )(X, X)


def _sc_gather_body(nchunks, rpw, xp_hbm, idx_hbm, g_hbm,
                    idx_v, rows_a, rows_b, sem):
    c = lax.axis_index("c")
    s = lax.axis_index("s")
    wid = s * NC + c
    base_row = wid * rpw
    pltpu.sync_copy(idx_hbm.at[pl.ds(base_row, rpw)], idx_v)

    def chunk_body(j, _):
        cp0 = pltpu.async_copy(xp_hbm.at[idx_v.at[2 * j]], rows_a, sem)
        cp1 = pltpu.async_copy(xp_hbm.at[idx_v.at[2 * j + 1]], rows_b, sem)
        cp0.wait()
        cp1.wait()
        out_base = base_row * IDX_PER_ROW + j * 2 * IDX_PER_ROW
        pltpu.sync_copy(rows_a, g_hbm.at[pl.ds(out_base, IDX_PER_ROW)])
        pltpu.sync_copy(rows_b, g_hbm.at[pl.ds(out_base + IDX_PER_ROW,
                                               IDX_PER_ROW)])
        return 0

    lax.fori_loop(0, nchunks, chunk_body, 0)


def _neighbor_gather(Xp, idx2d, npad):
    Cp = Xp.shape[1]
    nrows = idx2d.shape[0]
    rpw = nrows // NW
    nchunks = rpw // 2
    mesh = plsc.VectorSubcoreMesh(core_axis_name="c", subcore_axis_name="s")
    f = pl.kernel(
        functools.partial(_sc_gather_body, nchunks, rpw),
        out_type=jax.ShapeDtypeStruct((K * npad, Cp), jnp.float32),
        mesh=mesh,
        scratch_types=[
            pltpu.VMEM((rpw, IDX_PER_ROW), jnp.int32),
            pltpu.VMEM((IDX_PER_ROW, Cp), jnp.float32),
            pltpu.VMEM((IDX_PER_ROW, Cp), jnp.float32),
            pltpu.SemaphoreType.DMA,
        ],
    )
    return f(Xp, idx2d)


def _conv_body(R, C, O, g_ref, x_ref, wt_ref, gam_ref, bet_ref, out_ref):
    G = g_ref[...][:, :, :C]
    xb = x_ref[...]
    ctr = jnp.broadcast_to(xb[None, :, :], (K, R, C))
    F = jnp.concatenate([ctr - G, ctr], axis=2)
    F16 = F.reshape(K * R, 2 * C).astype(jnp.bfloat16)
    w16 = wt_ref[...].astype(jnp.bfloat16)
    out = lax.dot_general(F16, w16, (((1,), (0,)), ((), ())),
                          preferred_element_type=jnp.float32)
    o2 = out / jnp.sqrt(jnp.float32(1.0 + EPS)) * gam_ref[...] + bet_ref[...]
    o2 = jnp.where(o2 >= 0, o2, 0.2 * o2)
    acc = o2[0:R]
    for k in range(1, K):
        acc = jnp.maximum(acc, o2[k * R:(k + 1) * R])
    out_ref[...] = acc


def _edge_conv(G3, X, W, gam, bet):
    n, C = X.shape
    O = W.shape[0]
    Cp = G3.shape[2]
    R = 200 if n % 200 == 0 else 8
    grid = (n // R,)
    return pl.pallas_call(
        functools.partial(_conv_body, R, C, O),
        grid=grid,
        in_specs=[pl.BlockSpec((K, R, Cp), lambda i: (0, i, 0)),
                  pl.BlockSpec((R, C), lambda i: (i, 0)),
                  pl.BlockSpec((2 * C, O), lambda i: (0, 0)),
                  pl.BlockSpec((1, O), lambda i: (0, 0)),
                  pl.BlockSpec((1, O), lambda i: (0, 0))],
        out_specs=pl.BlockSpec((R, O), lambda i: (i, 0)),
        out_shape=jax.ShapeDtypeStruct((n, O), jnp.float32),
    )(G3, X, W.T, gam[None, :], bet[None, :])


def _dense_body(xc_ref, w5t_ref, g5_ref, b5_ref, wft_ref, gf_ref, bf_ref,
                wsemt_ref, bsem_ref, f_ref, sem_ref):
    rt = jnp.sqrt(jnp.float32(1.0 + EPS))
    xc16 = xc_ref[...].astype(jnp.bfloat16)
    w5t16 = w5t_ref[...].astype(jnp.bfloat16)
    x5 = lax.dot_general(xc16, w5t16, (((1,), (0,)), ((), ())),
                         preferred_element_type=jnp.float32)
    x5 = x5 / rt * g5_ref[...] + b5_ref[...]
    x5 = jnp.where(x5 >= 0, x5, 0.2 * x5)
    x516 = x5.astype(jnp.bfloat16)
    wft16 = wft_ref[...].astype(jnp.bfloat16)
    f = lax.dot_general(x516, wft16, (((1,), (0,)), ((), ())),
                        preferred_element_type=jnp.float32)
    f = f / rt * gf_ref[...] + bf_ref[...]
    f_ref[...] = f
    f316 = f[:, :256].astype(jnp.bfloat16)
    wsemt16 = wsemt_ref[...].astype(jnp.bfloat16)
    sem = lax.dot_general(f316, wsemt16, (((1,), (0,)), ((), ())),
                          preferred_element_type=jnp.float32)
    sem_ref[...] = sem + bsem_ref[...]


def _dense_head(Xc, w5t, g5r, b5r, wft, gfr, bfr, wsemt, bsemr):
    n = Xc.shape[0]
    RB = 1000 if n % 1000 == 0 else 8
    grid = (n // RB,)
    return pl.pallas_call(
        _dense_body,
        grid=grid,
        in_specs=[pl.BlockSpec((RB, 512), lambda i: (i, 0)),
                  pl.BlockSpec((512, 1024), lambda i: (0, 0)),
                  pl.BlockSpec((1, 1024), lambda i: (0, 0)),
                  pl.BlockSpec((1, 1024), lambda i: (0, 0)),
                  pl.BlockSpec((1024, 448), lambda i: (0, 0)),
                  pl.BlockSpec((1, 448), lambda i: (0, 0)),
                  pl.BlockSpec((1, 448), lambda i: (0, 0)),
                  pl.BlockSpec((256, 20), lambda i: (0, 0)),
                  pl.BlockSpec((1, 20), lambda i: (0, 0))],
        out_specs=[pl.BlockSpec((RB, 448), lambda i: (i, 0)),
                   pl.BlockSpec((RB, 20), lambda i: (i, 0))],
        out_shape=[jax.ShapeDtypeStruct((n, 448), jnp.float32),
                   jax.ShapeDtypeStruct((n, 20), jnp.float32)],
    )(Xc, w5t, g5r, b5r, wft, gfr, bfr, wsemt, bsemr)


def _edge_layer(X, W, g, b, npad):
    n, C = X.shape
    idx = _topk(X)
    idx_p = jnp.pad(idx, ((0, npad - n), (0, 0)))
    idx_km = idx_p.T.reshape(K * npad // IDX_PER_ROW, IDX_PER_ROW)
    Cp = max(128, C)
    Xp = jnp.pad(X, ((0, 0), (0, Cp - C))) if Cp != C else X
    G = _neighbor_gather(Xp, idx_km, npad)
    G3 = G.reshape(K, npad, Cp)
    return _edge_conv(G3, X, W, g, b)


def kernel(pt_coord, feats, W1, g1, b1, W2, g2, b2, W3, g3, b3, W4, g4, b4,
           W5, g5, b5, Wf1, gf1, bf1, Wf2, gf2, bf2, Wf3, gf3, bf3, Wsem, bsem):
    B, n, _ = pt_coord.shape
    npad = ((n + 1023) // 1024) * 1024

    ms1, ms2, ms3, sems = [], [], [], []
    for bi in range(B):
        X0 = jnp.concatenate([pt_coord[bi], feats[bi][:, 3:4]], axis=1)
        X1 = _edge_layer(X0, W1, g1, b1, npad)
        X2 = _edge_layer(X1, W2, g2, b2, npad)
        X3 = _edge_layer(X2, W3, g3, b3, npad)
        X4 = _edge_layer(X3, W4, g4, b4, npad)
        Xc = jnp.concatenate([X1, X2, X3, X4], axis=1)

        gfr = jnp.concatenate([gf3, gf1, gf2])[None, :]
        bfr = jnp.concatenate([bf3, bf1, bf2])[None, :]
        wft = jnp.concatenate([Wf3, Wf1, Wf2], axis=0).T
        F, sem = _dense_head(Xc, W5.T, g5[None, :], b5[None, :],
                             wft, gfr, bfr, Wsem.T, bsem[None, :])
        ms3.append(F[:, :256])
        ms1.append(F[:, 256:320])
        ms2.append(F[:, 320:448])
        sems.append(sem)

    ms_features = (jnp.stack(ms1), jnp.stack(ms2), jnp.stack(ms3))
    masks = jnp.zeros((B, n), dtype=bool)
    return (ms_features, pt_coord, masks, jnp.stack(sems))
